# baseline (speedup 1.0000x reference)
import functools

import jax
import jax.numpy as jnp
from jax import lax
from jax.experimental import pallas as pl
from jax.experimental.pallas import tpu as pltpu
from jax.experimental.pallas import tpu_sc as plsc

_E_DIM = 32
_BETA = 0.25

_T_BLK = 1024
_C_BLK = 1024
_SUB = 128
_I32_MAX = 2147483647


def _dist_body(zb, qc, c2, minv, mini):
    zt = zb[...].reshape(_E_DIM, _T_BLK)
    z2 = 0.5 * jnp.sum(zt * zt, axis=0, keepdims=True)
    rv = jnp.full((_SUB, _T_BLK), jnp.inf, jnp.float32)
    rk = jnp.zeros((_SUB, _T_BLK), jnp.int32)
    n_j = _C_BLK // _SUB
    for s in range(qc.shape[0] // _C_BLK):
        s0 = s * _C_BLK
        m = lax.dot_general(qc[s0:s0 + _C_BLK, :], zt,
                            (((1,), (0,)), ((), ())),
                            preferred_element_type=jnp.float32)
        for j in range(n_j):
            r0 = j * _SUB
            u = (z2 + c2[s0 + r0:s0 + r0 + _SUB, :]) - m[r0:r0 + _SUB, :]
            better = u < rv
            rv = jnp.where(better, u, rv)
            rk = jnp.where(better, s * n_j + j, rk)
    bmin = jnp.min(rv, axis=0, keepdims=True)
    comp = rk * _SUB + lax.broadcasted_iota(jnp.int32, rk.shape, 0)
    bidx = jnp.min(jnp.where(rv <= bmin, comp, _I32_MAX),
                   axis=0, keepdims=True)
    minv[...] = bmin.reshape(1, 1, _T_BLK)
    mini[...] = bidx.reshape(1, 1, _T_BLK)


def _dist_argmin(zr, qc, c2c):
    b = zr.shape[0]
    n_e = qc.shape[0]
    return pl.pallas_call(
        _dist_body,
        grid=(b,),
        in_specs=[
            pl.BlockSpec((1, _E_DIM, _T_BLK), lambda t: (t, 0, 0)),
            pl.BlockSpec((n_e, _E_DIM), lambda t: (0, 0)),
            pl.BlockSpec((n_e, 1), lambda t: (0, 0)),
        ],
        out_specs=[
            pl.BlockSpec((1, 1, _T_BLK), lambda t: (t, 0, 0)),
            pl.BlockSpec((1, 1, _T_BLK), lambda t: (t, 0, 0)),
        ],
        out_shape=[
            jax.ShapeDtypeStruct((b, 1, _T_BLK), jnp.float32),
            jax.ShapeDtypeStruct((b, 1, _T_BLK), jnp.int32),
        ],
        compiler_params=pltpu.CompilerParams(
            dimension_semantics=("parallel",)),
    )(zr, qc, c2c)


def _sc_gather(table, idx):
    info = plsc.get_sparse_core_info()
    nc, ns = info.num_cores, info.num_subcores
    nw = nc * ns
    b = idx.shape[0]
    d = table.shape[1]
    ch = 128
    per_w = b // nw
    k = per_w // ch
    idx3 = idx.reshape(nw, k, ch)
    mesh = plsc.VectorSubcoreMesh(core_axis_name="c", subcore_axis_name="s")

    @functools.partial(
        pl.kernel, mesh=mesh,
        out_type=jax.ShapeDtypeStruct((b, d), jnp.float32),
        scratch_types=[
            pltpu.VMEM((k, ch), jnp.int32),
            pltpu.VMEM((per_w, d), jnp.float32),
            pltpu.SemaphoreType.DMA,
        ],
    )
    def g(table_hbm, idx_hbm, out_hbm, idx_v, rows_v, sem):
        wid = lax.axis_index("s") * nc + lax.axis_index("c")
        pltpu.sync_copy(idx_hbm.at[wid], idx_v)
        cps = [pltpu.async_copy(table_hbm.at[idx_v.at[j]],
                                rows_v.at[pl.ds(j * ch, ch)], sem)
               for j in range(k)]
        for cp in cps:
            cp.wait()
        pltpu.sync_copy(rows_v, out_hbm.at[pl.ds(wid * per_w, per_w)])

    return g(table, idx3)


def kernel(z, emb_weight, proj_W, proj_b):
    b, cdim, h, w = z.shape
    zr = z.reshape(b, cdim, h * w)
    quant_codebook = emb_weight @ proj_W.T + proj_b
    c2 = jnp.sum(quant_codebook ** 2, axis=1)

    minv, mini = _dist_argmin(zr, quant_codebook,
                              (0.5 * c2).reshape(-1, 1))
    loss_sum = jnp.sum(minv)
    idx = mini.reshape(-1)
    qc_pad = jnp.pad(quant_codebook, ((0, 0), (0, 128 - _E_DIM)))
    z_q_flat = _sc_gather(qc_pad, idx)[:, :_E_DIM]

    z_q_out = jnp.transpose(z_q_flat.reshape(b, h, w, cdim), (0, 3, 1, 2))
    idx_out = mini.reshape(b, h, w)
    n_elems = b * h * w * cdim
    commit_loss = (_BETA + 1.0) * (2.0 * loss_sum) / n_elems
    zero = jnp.zeros((), dtype=jnp.float32)
    return ((z_q_out, zero, idx_out), (zero, zero, commit_loss, zero))
